# baseline (speedup 1.0000x reference)
import functools

import jax
import jax.numpy as jnp
from jax import lax
from jax.experimental import pallas as pl
from jax.experimental.pallas import tpu as pltpu
from jax.experimental.pallas import tpu_sc as plsc

N = 10000
F = 128
H = 256
C = 40
E = 320000

NP = 10240
NC, NS = 2, 16
WORKERS = NC * NS
CH = 128
G = 8
NB0, NB1 = 11, 11
NB = max(NB0, NB1)
M = NB * G
EP = WORKERS * M * CH
RPT = NP // NS

_mesh = plsc.VectorSubcoreMesh(
    core_axis_name="c", subcore_axis_name="s", num_cores=NC, num_subcores=NS)


def _deg_body(dst_hbm, deg_hbm, idx_v, ones_v, zb_v, deg_sh):
    c = lax.axis_index("c")
    s = lax.axis_index("s")
    w = c * NS + s
    nb_c = jnp.where(c == 0, NB0, NB1)

    def _fill(i, _):
        zb_v[pl.ds(i * 16, 16)] = jnp.zeros((16,), jnp.float32)
        return 0
    lax.fori_loop(0, RPT // 16, _fill, 0)

    def _fill1(i, _):
        ones_v[pl.ds(i * 16, 16)] = jnp.ones((16,), jnp.float32)
        return 0
    lax.fori_loop(0, CH // 16, _fill1, 0)

    pltpu.sync_copy(zb_v, deg_sh.at[pl.ds(s * RPT, RPT)])
    plsc.subcore_barrier()

    for n in range(NB):
        @pl.when(n < nb_c)
        def _():
            pltpu.sync_copy(dst_hbm.at[w, pl.ds(n * G, G)], idx_v)

            def _scat(j, _):
                pltpu.sync_copy(ones_v, deg_sh.at[idx_v.at[j]], add=True)
                return 0
            lax.fori_loop(0, G, _scat, 0)

    plsc.subcore_barrier()
    pltpu.sync_copy(deg_sh.at[pl.ds(s * RPT, RPT)],
                    deg_hbm.at[c, pl.ds(s * RPT, RPT)])


_deg = pl.kernel(
    _deg_body,
    out_type=jax.ShapeDtypeStruct((NC, NP), jnp.float32),
    mesh=_mesh,
    scratch_types=[
        pltpu.VMEM((G, CH), jnp.int32),
        pltpu.VMEM((CH,), jnp.float32),
        pltpu.VMEM((RPT,), jnp.float32),
        pltpu.VMEM_SHARED((NP,), jnp.float32),
    ],
)


def _prop_body(src_hbm, dst_hbm, table_hbm, out_hbm, sidx_v, didx_v, rows_v,
               acc_sh, gsem0, gsem1):
    c = lax.axis_index("c")
    s = lax.axis_index("s")
    w = c * NS + s
    nb_c = jnp.where(c == 0, NB0, NB1)

    buf0 = rows_v.at[0]
    buf1 = rows_v.at[1]

    def _zrow(i, _):
        for k in range(F // 16):
            buf0[i, pl.ds(k * 16, 16)] = jnp.zeros((16,), jnp.float32)
        return 0
    lax.fori_loop(0, CH, _zrow, 0)

    def _zacc(k, _):
        pltpu.sync_copy(buf0, acc_sh.at[pl.ds(s * RPT + k * CH, CH)])
        return 0
    lax.fori_loop(0, RPT // CH, _zacc, 0)
    plsc.subcore_barrier()

    for n in range(NB):
        @pl.when(n < nb_c)
        def _():
            pltpu.sync_copy(src_hbm.at[w, pl.ds(n * G, G)], sidx_v)
            pltpu.sync_copy(dst_hbm.at[w, pl.ds(n * G, G)], didx_v)

            def _pair(p, _):
                j0 = 2 * p
                j1 = j0 + 1
                pltpu.async_copy(table_hbm.at[sidx_v.at[j0]], buf0, gsem0)
                pltpu.async_copy(table_hbm.at[sidx_v.at[j1]], buf1, gsem1)
                pltpu.make_async_copy(table_hbm.at[sidx_v.at[j0]], buf0,
                                      gsem0).wait()
                pltpu.sync_copy(buf0, acc_sh.at[didx_v.at[j0]], add=True)
                pltpu.make_async_copy(table_hbm.at[sidx_v.at[j1]], buf1,
                                      gsem1).wait()
                pltpu.sync_copy(buf1, acc_sh.at[didx_v.at[j1]], add=True)
                return 0
            lax.fori_loop(0, G // 2, _pair, 0)

    plsc.subcore_barrier()

    def _wout(k, _):
        r0 = s * RPT + k * CH
        pltpu.sync_copy(acc_sh.at[pl.ds(r0, CH)], out_hbm.at[c, pl.ds(r0, CH)])
        return 0
    lax.fori_loop(0, RPT // CH, _wout, 0)


_prop = pl.kernel(
    _prop_body,
    out_type=jax.ShapeDtypeStruct((NC, NP, F), jnp.float32),
    mesh=_mesh,
    scratch_types=[
        pltpu.VMEM((G, CH), jnp.int32),
        pltpu.VMEM((G, CH), jnp.int32),
        pltpu.VMEM((2, CH, F), jnp.float32),
        pltpu.VMEM_SHARED((NP, F), jnp.float32),
        pltpu.SemaphoreType.DMA,
        pltpu.SemaphoreType.DMA,
    ],
)

_R = 2048


def _pre_body(deg_ref, x_ref, dinv_ref, x1_ref):
    d = deg_ref[0] + deg_ref[1]
    dinv = lax.rsqrt(jnp.maximum(d, 1.0))
    dinv_ref[...] = dinv
    x1_ref[...] = x_ref[...] * dinv


_pre = pl.pallas_call(
    _pre_body,
    grid=(NP // _R,),
    in_specs=[
        pl.BlockSpec((NC, _R, 1), lambda i: (0, i, 0)),
        pl.BlockSpec((_R, F), lambda i: (i, 0)),
    ],
    out_specs=[
        pl.BlockSpec((_R, 1), lambda i: (i, 0)),
        pl.BlockSpec((_R, F), lambda i: (i, 0)),
    ],
    out_shape=[
        jax.ShapeDtypeStruct((NP, 1), jnp.float32),
        jax.ShapeDtypeStruct((NP, F), jnp.float32),
    ],
)


def _mid_body(y1p_ref, dinv_ref, w1_ref, w2_ref, g_ref):
    dinv = dinv_ref[...]
    y1 = (y1p_ref[0] + y1p_ref[1]) * dinv
    h1 = jnp.maximum(jnp.dot(y1, w1_ref[...], preferred_element_type=jnp.float32), 0.0)
    g = jnp.dot(h1, w2_ref[...], preferred_element_type=jnp.float32) * dinv
    g_ref[0] = g[:, :F]
    g_ref[1] = g[:, F:]


_mid = pl.pallas_call(
    _mid_body,
    grid=(NP // _R,),
    in_specs=[
        pl.BlockSpec((NC, _R, F), lambda i: (0, i, 0)),
        pl.BlockSpec((_R, 1), lambda i: (i, 0)),
        pl.BlockSpec((F, H), lambda i: (0, 0)),
        pl.BlockSpec((H, H), lambda i: (0, 0)),
    ],
    out_specs=pl.BlockSpec((2, _R, F), lambda i: (0, i, 0)),
    out_shape=jax.ShapeDtypeStruct((2, NP, F), jnp.float32),
)


def _fin_body(y2a_ref, y2b_ref, wx_ref, wy_ref, z_ref, xh_ref, yh_ref):
    h2a = y2a_ref[0] + y2a_ref[1]
    h2b = y2b_ref[0] + y2b_ref[1]
    h2 = jnp.concatenate([h2a, h2b], axis=1)
    nrm = jnp.sqrt(jnp.sum(h2 * h2, axis=1, keepdims=True))
    z = h2 / jnp.maximum(nrm, 1e-12)
    z_ref[...] = z
    xh_ref[...] = jnp.dot(z, wx_ref[...], preferred_element_type=jnp.float32)
    yh_ref[...] = jnp.dot(z, wy_ref[...], preferred_element_type=jnp.float32)


_fin = pl.pallas_call(
    _fin_body,
    grid=(NP // _R,),
    in_specs=[
        pl.BlockSpec((NC, _R, F), lambda i: (0, i, 0)),
        pl.BlockSpec((NC, _R, F), lambda i: (0, i, 0)),
        pl.BlockSpec((H, F), lambda i: (0, 0)),
        pl.BlockSpec((H, C), lambda i: (0, 0)),
    ],
    out_specs=[
        pl.BlockSpec((_R, H), lambda i: (i, 0)),
        pl.BlockSpec((_R, F), lambda i: (i, 0)),
        pl.BlockSpec((_R, C), lambda i: (i, 0)),
    ],
    out_shape=[
        jax.ShapeDtypeStruct((NP, H), jnp.float32),
        jax.ShapeDtypeStruct((NP, F), jnp.float32),
        jax.ShapeDtypeStruct((NP, C), jnp.float32),
    ],
)


REAL = E + N
CAP0 = NS * NB0 * G * CH
CAP1 = NS * NB1 * G * CH
SP = max(REAL - CAP1, min(CAP0, REAL * NB0 // (NB0 + NB1)))


def _arrange(v_real, spread_pads):
    blocks = []
    for nb, lo, hi in ((NB0, 0, SP), (NB1, SP, REAL)):
        cap = NS * nb * G * CH
        if nb == 0:
            blocks.append(jnp.zeros((NS, M, CH), jnp.int32))
            continue
        v = v_real[lo:hi]
        npad = cap - (hi - lo)
        if spread_pads:
            pad = N + jnp.arange(npad, dtype=jnp.int32) % (NP - N)
        else:
            pad = jnp.full((npad,), N, dtype=jnp.int32)
        b = jnp.concatenate([v, pad]).reshape(NS, nb * G, CH)
        blocks.append(jnp.pad(b, ((0, 0), (0, (NB - nb) * G), (0, 0))))
    return jnp.concatenate(blocks, axis=0)


def kernel(edge_index, x, W1, W2, Wx, Wy):
    loop = jnp.arange(N, dtype=jnp.int32)
    srcp = _arrange(jnp.concatenate([edge_index[0], loop]), False)
    dstp = _arrange(jnp.concatenate([edge_index[1], loop]), True)
    xp = jnp.pad(x, ((0, NP - N), (0, 0)))

    deg2 = _deg(dstp)
    dinv, x1 = _pre(deg2.reshape(NC, NP, 1), xp)
    y1p = _prop(srcp, dstp, x1)
    g = _mid(y1p, dinv, W1, W2)
    y2a = _prop(srcp, dstp, g[0])
    y2b = _prop(srcp, dstp, g[1])
    z, xh, yh = _fin(y2a, y2b, Wx, Wy)
    return z[:N], xh[:N], yh[:N]


# speedup vs baseline: 2.5070x; 2.5070x over previous
import functools

import jax
import jax.numpy as jnp
from jax import lax
from jax.experimental import pallas as pl
from jax.experimental.pallas import tpu as pltpu
from jax.experimental.pallas import tpu_sc as plsc

N = 10000
F = 128
H = 256
C = 40
E = 320000

NP = 10240
NC, NS = 2, 16
WORKERS = NC * NS
CH = 128
M = 82
EP = WORKERS * M * CH
RPT = NP // NS

_mesh = plsc.VectorSubcoreMesh(
    core_axis_name="c", subcore_axis_name="s", num_cores=NC, num_subcores=NS)


def _deg_body(dst_hbm, deg_hbm, idx_v, ones_v, zb_v, deg_sh):
    c = lax.axis_index("c")
    s = lax.axis_index("s")
    w = c * NS + s
    pltpu.sync_copy(dst_hbm.at[w], idx_v)

    def _fill(i, _):
        zb_v[pl.ds(i * 16, 16)] = jnp.zeros((16,), jnp.float32)
        return 0
    lax.fori_loop(0, RPT // 16, _fill, 0)

    def _fill1(i, _):
        ones_v[pl.ds(i * 16, 16)] = jnp.ones((16,), jnp.float32)
        return 0
    lax.fori_loop(0, CH // 16, _fill1, 0)

    pltpu.sync_copy(zb_v, deg_sh.at[pl.ds(s * RPT, RPT)])
    plsc.subcore_barrier()

    def _scat(j, _):
        pltpu.sync_copy(ones_v, deg_sh.at[idx_v.at[j]], add=True)
        return 0
    lax.fori_loop(0, M, _scat, 0)

    plsc.subcore_barrier()
    pltpu.sync_copy(deg_sh.at[pl.ds(s * RPT, RPT)],
                    deg_hbm.at[c, pl.ds(s * RPT, RPT)])


_deg = pl.kernel(
    _deg_body,
    out_type=jax.ShapeDtypeStruct((NC, NP), jnp.float32),
    mesh=_mesh,
    scratch_types=[
        pltpu.VMEM((M, CH), jnp.int32),
        pltpu.VMEM((CH,), jnp.float32),
        pltpu.VMEM((RPT,), jnp.float32),
        pltpu.VMEM_SHARED((NP,), jnp.float32),
    ],
)


def _prop_body(src_hbm, dst_hbm, table_hbm, out_hbm, sidx_v, didx_v, rows_v,
               acc_sh, gsem):
    c = lax.axis_index("c")
    s = lax.axis_index("s")
    w = c * NS + s
    pltpu.sync_copy(src_hbm.at[w], sidx_v)
    pltpu.sync_copy(dst_hbm.at[w], didx_v)

    def _zrow(i, _):
        for k in range(F // 16):
            rows_v[i, pl.ds(k * 16, 16)] = jnp.zeros((16,), jnp.float32)
        return 0
    lax.fori_loop(0, CH, _zrow, 0)

    def _zacc(k, _):
        pltpu.sync_copy(rows_v, acc_sh.at[pl.ds(s * RPT + k * CH, CH)])
        return 0
    lax.fori_loop(0, RPT // CH, _zacc, 0)
    plsc.subcore_barrier()

    def _edge(j, _):
        pltpu.async_copy(table_hbm.at[sidx_v.at[j]], rows_v, gsem).wait()
        pltpu.sync_copy(rows_v, acc_sh.at[didx_v.at[j]], add=True)
        return 0
    lax.fori_loop(0, M, _edge, 0)

    plsc.subcore_barrier()

    def _wout(k, _):
        r0 = s * RPT + k * CH
        pltpu.sync_copy(acc_sh.at[pl.ds(r0, CH)], out_hbm.at[c, pl.ds(r0, CH)])
        return 0
    lax.fori_loop(0, RPT // CH, _wout, 0)


_prop = pl.kernel(
    _prop_body,
    out_type=jax.ShapeDtypeStruct((NC, NP, F), jnp.float32),
    mesh=_mesh,
    scratch_types=[
        pltpu.VMEM((M, CH), jnp.int32),
        pltpu.VMEM((M, CH), jnp.int32),
        pltpu.VMEM((CH, F), jnp.float32),
        pltpu.VMEM_SHARED((NP, F), jnp.float32),
        pltpu.SemaphoreType.DMA,
    ],
)

_R = 2048


def _pre_body(deg_ref, x_ref, dinv_ref, x1_ref):
    d = deg_ref[0] + deg_ref[1]
    dinv = lax.rsqrt(jnp.maximum(d, 1.0))
    dinv_ref[...] = dinv
    x1_ref[...] = x_ref[...] * dinv


_pre = pl.pallas_call(
    _pre_body,
    grid=(NP // _R,),
    in_specs=[
        pl.BlockSpec((NC, _R, 1), lambda i: (0, i, 0)),
        pl.BlockSpec((_R, F), lambda i: (i, 0)),
    ],
    out_specs=[
        pl.BlockSpec((_R, 1), lambda i: (i, 0)),
        pl.BlockSpec((_R, F), lambda i: (i, 0)),
    ],
    out_shape=[
        jax.ShapeDtypeStruct((NP, 1), jnp.float32),
        jax.ShapeDtypeStruct((NP, F), jnp.float32),
    ],
)


def _mid_body(y1p_ref, dinv_ref, w1_ref, w2_ref, g_ref):
    dinv = dinv_ref[...]
    y1 = (y1p_ref[0] + y1p_ref[1]) * dinv
    h1 = jnp.maximum(jnp.dot(y1, w1_ref[...], preferred_element_type=jnp.float32), 0.0)
    g = jnp.dot(h1, w2_ref[...], preferred_element_type=jnp.float32) * dinv
    g_ref[0] = g[:, :F]
    g_ref[1] = g[:, F:]


_mid = pl.pallas_call(
    _mid_body,
    grid=(NP // _R,),
    in_specs=[
        pl.BlockSpec((NC, _R, F), lambda i: (0, i, 0)),
        pl.BlockSpec((_R, 1), lambda i: (i, 0)),
        pl.BlockSpec((F, H), lambda i: (0, 0)),
        pl.BlockSpec((H, H), lambda i: (0, 0)),
    ],
    out_specs=pl.BlockSpec((2, _R, F), lambda i: (0, i, 0)),
    out_shape=jax.ShapeDtypeStruct((2, NP, F), jnp.float32),
)


def _fin_body(y2a_ref, y2b_ref, wx_ref, wy_ref, z_ref, xh_ref, yh_ref):
    h2a = y2a_ref[0] + y2a_ref[1]
    h2b = y2b_ref[0] + y2b_ref[1]
    h2 = jnp.concatenate([h2a, h2b], axis=1)
    nrm = jnp.sqrt(jnp.sum(h2 * h2, axis=1, keepdims=True))
    z = h2 / jnp.maximum(nrm, 1e-12)
    z_ref[...] = z
    xh_ref[...] = jnp.dot(z, wx_ref[...], preferred_element_type=jnp.float32)
    yh_ref[...] = jnp.dot(z, wy_ref[...], preferred_element_type=jnp.float32)


_fin = pl.pallas_call(
    _fin_body,
    grid=(NP // _R,),
    in_specs=[
        pl.BlockSpec((NC, _R, F), lambda i: (0, i, 0)),
        pl.BlockSpec((NC, _R, F), lambda i: (0, i, 0)),
        pl.BlockSpec((H, F), lambda i: (0, 0)),
        pl.BlockSpec((H, C), lambda i: (0, 0)),
    ],
    out_specs=[
        pl.BlockSpec((_R, H), lambda i: (i, 0)),
        pl.BlockSpec((_R, F), lambda i: (i, 0)),
        pl.BlockSpec((_R, C), lambda i: (i, 0)),
    ],
    out_shape=[
        jax.ShapeDtypeStruct((NP, H), jnp.float32),
        jax.ShapeDtypeStruct((NP, F), jnp.float32),
        jax.ShapeDtypeStruct((NP, C), jnp.float32),
    ],
)


def kernel(edge_index, x, W1, W2, Wx, Wy):
    loop = jnp.arange(N, dtype=jnp.int32)
    npad = EP - E - N
    pad_src = jnp.full((npad,), N, dtype=jnp.int32)
    pad_dst = N + jnp.arange(npad, dtype=jnp.int32) % (NP - N)
    srcp = jnp.concatenate([edge_index[0], loop, pad_src]).reshape(WORKERS, M, CH)
    dstp = jnp.concatenate([edge_index[1], loop, pad_dst]).reshape(WORKERS, M, CH)
    xp = jnp.pad(x, ((0, NP - N), (0, 0)))

    deg2 = _deg(dstp)
    dinv, x1 = _pre(deg2.reshape(NC, NP, 1), xp)
    y1p = _prop(srcp, dstp, x1)
    g = _mid(y1p, dinv, W1, W2)
    y2a = _prop(srcp, dstp, g[0])
    y2b = _prop(srcp, dstp, g[1])
    z, xh, yh = _fin(y2a, y2b, Wx, Wy)
    return z[:N], xh[:N], yh[:N]
